# baseline (speedup 1.0000x reference)
import functools

import jax
import jax.numpy as jnp
from jax import lax
from jax.experimental import pallas as pl
from jax.experimental.pallas import tpu as pltpu
from jax.experimental.pallas import tpu_sc as plsc

_G = 2
_V = 320
_GV = _G * _V

_BM = 1024
_CH = 128
_HALF = 8192

_KS0 = 0
_KS1 = 42
_KS2 = 0x1BD11BDA ^ _KS0 ^ _KS1
_ROTS = ((13, 15, 26, 6), (17, 29, 16, 24))


def _uniform_042(p):
    ks = (jnp.uint32(_KS0), jnp.uint32(_KS1), jnp.uint32(_KS2))
    x0 = jnp.full_like(p, ks[0])
    x1 = p + ks[1]
    for r in range(5):
        for rot in _ROTS[r % 2]:
            x0 = x0 + x1
            x1 = (x1 << jnp.uint32(rot)) | (x1 >> jnp.uint32(32 - rot))
            x1 = x1 ^ x0
        x0 = x0 + ks[(r + 1) % 3]
        x1 = x1 + ks[(r + 2) % 3] + jnp.uint32(r + 1)
    bits = x0 ^ x1
    f = lax.bitcast_convert_type(
        (bits >> jnp.uint32(9)) | jnp.uint32(0x3F800000), jnp.float32)
    f = f - jnp.float32(1.0)
    return jnp.maximum(jnp.float32(1e-10),
                       f * jnp.float32(1.0 - 1e-10) + jnp.float32(1e-10))


def _tc_body(wt_ref, hs_ref, bt_ref, acc0_ref, idx0_ref, idx1_ref,
             accum_ref, perp_ref, *, tok_base, n_rows, n_steps, last_phase):
    i = pl.program_id(0)
    ht = lax.dot_general(wt_ref[...], hs_ref[...],
                         (((1,), (1,)), ((), ())),
                         preferred_element_type=jnp.float32) + bt_ref[...]
    rowc = lax.broadcasted_iota(jnp.int32, ht.shape, 0)
    lanet = lax.broadcasted_iota(jnp.int32, ht.shape, 1)
    p = ((tok_base + i * _BM + lanet) * _GV + rowc).astype(jnp.uint32)
    u = _uniform_042(p)
    z = ht - jnp.log(-jnp.log(u))
    g0 = rowc < _V
    neg = jnp.float32(-jnp.inf)
    m0 = jnp.max(jnp.where(g0, z, neg), axis=0, keepdims=True)
    m1 = jnp.max(jnp.where(g0, neg, z), axis=0, keepdims=True)
    idx0 = jnp.min(jnp.where(g0 & (z == m0), rowc, _GV), axis=0,
                   keepdims=True)
    idx1 = jnp.min(jnp.where((~g0) & (z == m1), rowc, _GV), axis=0,
                   keepdims=True)
    idx0_ref[...] = idx0.reshape(1, _BM // 128, 128).astype(jnp.int32)
    idx1_ref[...] = idx1.reshape(1, _BM // 128, 128).astype(jnp.int32)

    hm0 = jnp.max(jnp.where(g0, ht, neg), axis=0, keepdims=True)
    hm1 = jnp.max(jnp.where(g0, neg, ht), axis=0, keepdims=True)
    e = jnp.exp(ht - jnp.where(g0, hm0, hm1))
    s0 = jnp.sum(jnp.where(g0, e, 0.0), axis=0, keepdims=True)
    s1 = jnp.sum(jnp.where(g0, 0.0, e), axis=0, keepdims=True)
    s = e / jnp.where(g0, s0, s1)
    rowsum = jnp.sum(s, axis=1, keepdims=True)

    @pl.when(i == 0)
    def _():
        accum_ref[...] = acc0_ref[...]

    accum_ref[...] += jnp.broadcast_to(rowsum, accum_ref.shape)

    if last_phase:
        @pl.when(i == n_steps - 1)
        def _():
            marg = accum_ref[...] / n_rows
            ent = marg * jnp.log(marg + 1e-7)
            gmask = lax.broadcasted_iota(jnp.int32, ent.shape, 0) < _V
            e0 = jnp.exp(-jnp.sum(jnp.where(gmask, ent, 0.0), axis=0,
                                  keepdims=True))
            e1 = jnp.exp(-jnp.sum(jnp.where(gmask, 0.0, ent), axis=0,
                                  keepdims=True))
            perp_ref[...] = e0 + e1
    else:
        @pl.when(i == n_steps - 1)
        def _():
            perp_ref[...] = jnp.zeros_like(perp_ref)


def _tc_call(hs_half, wt, bt, acc0, tok_base, n_rows, last_phase):
    m, d = hs_half.shape
    n_steps = m // _BM
    return pl.pallas_call(
        functools.partial(_tc_body, tok_base=tok_base, n_rows=n_rows,
                          n_steps=n_steps, last_phase=last_phase),
        grid=(n_steps,),
        in_specs=[
            pl.BlockSpec((_GV, d), lambda i: (0, 0)),
            pl.BlockSpec((_BM, d), lambda i: (i, 0)),
            pl.BlockSpec((_GV, _BM), lambda i: (0, 0)),
            pl.BlockSpec((_GV, 128), lambda i: (0, 0)),
        ],
        out_specs=[
            pl.BlockSpec((1, _BM // 128, 128), lambda i: (i, 0, 0)),
            pl.BlockSpec((1, _BM // 128, 128), lambda i: (i, 0, 0)),
            pl.BlockSpec((_GV, 128), lambda i: (0, 0)),
            pl.BlockSpec((1, 128), lambda i: (0, 0)),
        ],
        out_shape=[
            jax.ShapeDtypeStruct((n_steps, _BM // 128, 128), jnp.int32),
            jax.ShapeDtypeStruct((n_steps, _BM // 128, 128), jnp.int32),
            jax.ShapeDtypeStruct((_GV, 128), jnp.float32),
            jax.ShapeDtypeStruct((1, 128), jnp.float32),
        ],
    )(wt, hs_half, bt, acc0)


def _sc_gather(table, idx0, idx1, out_ref, tok_base):
    info = plsc.get_sparse_core_info()
    nc, ns = info.num_cores, info.num_subcores
    nw = nc * ns
    d = table.shape[1]
    tpw = _HALF // nw
    nch = tpw // _CH
    mesh = plsc.VectorSubcoreMesh(core_axis_name="c", subcore_axis_name="s")

    @functools.partial(
        pl.kernel, mesh=mesh,
        out_type=(),
        scratch_types=[
            pltpu.VMEM((_CH, d), jnp.float32),
            pltpu.VMEM((_CH, d), jnp.float32),
            pltpu.VMEM((nch, _CH), jnp.int32),
            pltpu.VMEM((nch, _CH), jnp.int32),
            pltpu.SemaphoreType.DMA,
            pltpu.SemaphoreType.DMA,
        ],
    )
    def k(table_hbm, idx0_hbm, idx1_hbm, out_hbm, rows0_v, rows1_v,
          idx0_v, idx1_v, sem0, sem1):
        wid = lax.axis_index("s") * nc + lax.axis_index("c")
        step = wid // 4
        r0 = nch * (wid % 4)
        tok0 = tok_base + wid * tpw
        pltpu.sync_copy(idx0_hbm.at[step, pl.ds(r0, nch)], idx0_v)
        pltpu.sync_copy(idx1_hbm.at[step, pl.ds(r0, nch)], idx1_v)
        for ch in range(nch):
            cp0 = pltpu.async_copy(
                table_hbm.at[idx0_v.at[ch]], rows0_v, sem0)
            cp1 = pltpu.async_copy(
                table_hbm.at[idx1_v.at[ch]], rows1_v, sem1)
            cp0.wait()
            pltpu.sync_copy(rows0_v,
                            out_hbm.at[pl.ds(tok0 + ch * _CH, _CH),
                                       pl.ds(0, d)])
            cp1.wait()
            pltpu.sync_copy(rows1_v,
                            out_hbm.at[pl.ds(tok0 + ch * _CH, _CH),
                                       pl.ds(d, d)])

    k(table, idx0, idx1, out_ref)


def kernel(hidden_states, W, b, codevectors):
    B, T, D = hidden_states.shape
    n = B * T
    dv = codevectors.shape[-1]
    hs2 = hidden_states.reshape(n, D)
    wt = W.T
    bt = jnp.broadcast_to(b.reshape(_GV, 1), (_GV, _BM))
    table = codevectors.reshape(_GV, dv)

    zeros = jnp.zeros((_GV, 128), jnp.float32)
    idx0a, idx1a, acc_a, _ = _tc_call(
        hs2[:_HALF], wt, bt, zeros, 0, n, False)
    buf = jax.new_ref(jnp.zeros((n, _G * dv), jnp.float32))
    _sc_gather(table, idx0a, idx1a, buf, 0)
    idx0b, idx1b, _acc_b, perp = _tc_call(
        hs2[_HALF:], wt, bt, acc_a, _HALF, n, True)
    _sc_gather(table, idx0b, idx1b, buf, _HALF)
    cv = buf[...]
    return cv.reshape(B, T, _G * dv), perp[0, 0].reshape(())


# speedup vs baseline: 1.1233x; 1.1233x over previous
import functools

import jax
import jax.numpy as jnp
from jax import lax
from jax.experimental import pallas as pl
from jax.experimental.pallas import tpu as pltpu
from jax.experimental.pallas import tpu_sc as plsc

_G = 2
_V = 320
_GV = _G * _V

_BM = 1024
_CH = 128

_KS0 = 0
_KS1 = 42
_KS2 = 0x1BD11BDA ^ _KS0 ^ _KS1
_ROTS = ((13, 15, 26, 6), (17, 29, 16, 24))


def _uniform_042(p):
    ks = (jnp.uint32(_KS0), jnp.uint32(_KS1), jnp.uint32(_KS2))
    x0 = jnp.full_like(p, ks[0])
    x1 = p + ks[1]
    for r in range(5):
        for rot in _ROTS[r % 2]:
            x0 = x0 + x1
            x1 = (x1 << jnp.uint32(rot)) | (x1 >> jnp.uint32(32 - rot))
            x1 = x1 ^ x0
        x0 = x0 + ks[(r + 1) % 3]
        x1 = x1 + ks[(r + 2) % 3] + jnp.uint32(r + 1)
    bits = x0 ^ x1
    f = lax.bitcast_convert_type(
        (bits >> jnp.uint32(9)) | jnp.uint32(0x3F800000), jnp.float32)
    f = f - jnp.float32(1.0)
    return jnp.maximum(jnp.float32(1e-10),
                       f * jnp.float32(1.0 - 1e-10) + jnp.float32(1e-10))


def _tc_body(w_ref, hs_ref, bt_ref, idx0_ref, idx1_ref, accum_ref, perp_ref,
             *, n_rows, n_steps):
    i = pl.program_id(0)
    ht = lax.dot_general(w_ref[...], hs_ref[...],
                         (((0,), (1,)), ((), ())),
                         preferred_element_type=jnp.float32) + bt_ref[...]
    rowc = lax.broadcasted_iota(jnp.int32, ht.shape, 0)
    lanet = lax.broadcasted_iota(jnp.int32, ht.shape, 1)
    p = ((i * _BM + lanet) * _GV + rowc).astype(jnp.uint32)
    u = _uniform_042(p)
    z = ht - jnp.log(-jnp.log(u))
    g0 = rowc < _V
    neg = jnp.float32(-jnp.inf)
    m0 = jnp.max(jnp.where(g0, z, neg), axis=0, keepdims=True)
    m1 = jnp.max(jnp.where(g0, neg, z), axis=0, keepdims=True)
    idx0 = jnp.min(jnp.where(g0 & (z == m0), rowc, _GV), axis=0,
                   keepdims=True)
    idx1 = jnp.min(jnp.where((~g0) & (z == m1), rowc, _GV), axis=0,
                   keepdims=True)
    idx0_ref[...] = idx0.reshape(1, _BM // 128, 128).astype(jnp.int32)
    idx1_ref[...] = idx1.reshape(1, _BM // 128, 128).astype(jnp.int32)

    hm0 = jnp.max(jnp.where(g0, ht, neg), axis=0, keepdims=True)
    hm1 = jnp.max(jnp.where(g0, neg, ht), axis=0, keepdims=True)
    e = jnp.exp(ht - jnp.where(g0, hm0, hm1))
    s0 = jnp.sum(jnp.where(g0, e, 0.0), axis=0, keepdims=True)
    s1 = jnp.sum(jnp.where(g0, 0.0, e), axis=0, keepdims=True)
    s = e / jnp.where(g0, s0, s1)
    rowsum = jnp.sum(s, axis=1, keepdims=True)

    @pl.when(i == 0)
    def _():
        accum_ref[...] = jnp.zeros_like(accum_ref)

    accum_ref[...] += jnp.broadcast_to(rowsum, accum_ref.shape)

    @pl.when(i == n_steps - 1)
    def _():
        marg = accum_ref[...] / n_rows
        ent = marg * jnp.log(marg + 1e-7)
        gmask = lax.broadcasted_iota(jnp.int32, ent.shape, 0) < _V
        e0 = jnp.exp(-jnp.sum(jnp.where(gmask, ent, 0.0), axis=0,
                              keepdims=True))
        e1 = jnp.exp(-jnp.sum(jnp.where(gmask, 0.0, ent), axis=0,
                              keepdims=True))
        perp_ref[...] = e0 + e1


def _tc_call(hs2, W, b):
    n, d = hs2.shape
    n_steps = n // _BM
    bt = jnp.broadcast_to(b.reshape(_GV, 1), (_GV, _BM))
    return pl.pallas_call(
        functools.partial(_tc_body, n_rows=n, n_steps=n_steps),
        grid=(n_steps,),
        in_specs=[
            pl.BlockSpec((d, _GV), lambda i: (0, 0)),
            pl.BlockSpec((_BM, d), lambda i: (i, 0)),
            pl.BlockSpec((_GV, _BM), lambda i: (0, 0)),
        ],
        out_specs=[
            pl.BlockSpec((1, _BM // 128, 128), lambda i: (i, 0, 0)),
            pl.BlockSpec((1, _BM // 128, 128), lambda i: (i, 0, 0)),
            pl.BlockSpec((_GV, 128), lambda i: (0, 0)),
            pl.BlockSpec((1, 128), lambda i: (0, 0)),
        ],
        out_shape=[
            jax.ShapeDtypeStruct((n_steps, _BM // 128, 128), jnp.int32),
            jax.ShapeDtypeStruct((n_steps, _BM // 128, 128), jnp.int32),
            jax.ShapeDtypeStruct((_GV, 128), jnp.float32),
            jax.ShapeDtypeStruct((1, 128), jnp.float32),
        ],
    )(W, hs2, bt)


def _sc_gather(table, idx0, idx1, B, T):
    info = plsc.get_sparse_core_info()
    nc, ns = info.num_cores, info.num_subcores
    nw = nc * ns
    n = B * T
    d = table.shape[1]
    tpw = n // nw
    nch = tpw // _CH
    rows_per_step = _BM // 128
    wpb = T // tpw
    mesh = plsc.VectorSubcoreMesh(core_axis_name="c", subcore_axis_name="s")

    @functools.partial(
        pl.kernel, mesh=mesh,
        out_type=jax.ShapeDtypeStruct((B, T, _G * d), jnp.float32),
        scratch_types=[
            pltpu.VMEM((2, _CH, d), jnp.float32),
            pltpu.VMEM((2, _CH, d), jnp.float32),
            pltpu.VMEM((4, _CH), jnp.int32),
            pltpu.VMEM((4, _CH), jnp.int32),
            [pltpu.SemaphoreType.DMA] * 4,
            [pltpu.SemaphoreType.DMA] * 4,
        ],
    )
    def k(table_hbm, idx0_hbm, idx1_hbm, out_hbm, rows0_v, rows1_v,
          idx0_v, idx1_v, gsems, wsems):
        wid = lax.axis_index("s") * nc + lax.axis_index("c")
        step = wid // 2
        half = wid % 2
        bi = wid // wpb
        t0 = (wid % wpb) * tpw
        r0 = (rows_per_step // 2) * half
        pltpu.sync_copy(idx0_hbm.at[step, pl.ds(r0, nch)], idx0_v)
        pltpu.sync_copy(idx1_hbm.at[step, pl.ds(r0, nch)], idx1_v)

        def gather(ch):
            sl = ch % 2
            return (pltpu.async_copy(table_hbm.at[idx0_v.at[ch]],
                                     rows0_v.at[sl], gsems[sl]),
                    pltpu.async_copy(table_hbm.at[idx1_v.at[ch]],
                                     rows1_v.at[sl], gsems[2 + sl]))

        def write(ch):
            sl = ch % 2
            dst = out_hbm.at[bi, pl.ds(t0 + ch * _CH, _CH)]
            return (pltpu.async_copy(rows0_v.at[sl], dst.at[:, pl.ds(0, d)],
                                     wsems[sl]),
                    pltpu.async_copy(rows1_v.at[sl], dst.at[:, pl.ds(d, d)],
                                     wsems[2 + sl]))

        cps = gather(0)
        wr = {}
        for ch in range(nch):
            if ch + 1 < nch:
                if ch >= 1:
                    for w in wr[ch - 1]:
                        w.wait()
                nxt = gather(ch + 1)
            for c in cps:
                c.wait()
            wr[ch] = write(ch)
            if ch + 1 < nch:
                cps = nxt
        for w in wr[nch - 2]:
            w.wait()
        for w in wr[nch - 1]:
            w.wait()

    return k(table, idx0, idx1)


def kernel(hidden_states, W, b, codevectors):
    B, T, D = hidden_states.shape
    n = B * T
    hs2 = hidden_states.reshape(n, D)
    idx0, idx1, _accum, perp = _tc_call(hs2, W, b)
    table = codevectors.reshape(_GV, codevectors.shape[-1])
    cv = _sc_gather(table, idx0, idx1, B, T)
    return cv, perp[0, 0].reshape(())
